# baseline (speedup 1.0000x reference)
import functools

import jax
import jax.numpy as jnp
from jax import lax
from jax.experimental import pallas as pl
from jax.experimental.pallas import tpu as pltpu
from jax.experimental.pallas import tpu_sc as plsc

B = 4
S = 2048
D = 2048
FLAT = B * S
NC = 2
NS = 16
NW = NC * NS
PW = S // NW
P = 2
CR = B * P
ROWS_PER_W = PW * B
NBUF = 2
NCHUNKS = PW // P
LANES = 16


def _body(xr_hbm, tok_hbm, pos_hbm, out_hbm,
          idx_v, tok_v, pos_v, out_v,
          sem_tok, sem_pos, sem_out):
    wid = lax.axis_index("s") * NC + lax.axis_index("c")
    base = wid * ROWS_PER_W
    pos0 = wid * PW

    pltpu.sync_copy(xr_hbm.at[pl.ds(base, ROWS_PER_W)], idx_v)

    def issue_in(c, b):
        pltpu.async_copy(
            tok_hbm.at[idx_v.at[pl.ds(c * CR, CR)]], tok_v.at[b], sem_tok[b])
        pltpu.async_copy(
            pos_hbm.at[pl.ds(pos0 + c * P, P)], pos_v.at[b], sem_pos[b])

    def wait_in(c, b):
        pltpu.make_async_copy(
            tok_hbm.at[idx_v.at[pl.ds(c * CR, CR)]], tok_v.at[b], sem_tok[b]).wait()
        pltpu.make_async_copy(
            pos_hbm.at[pl.ds(pos0 + c * P, P)], pos_v.at[b], sem_pos[b]).wait()

    def issue_out(c, b):
        for bb in range(B):
            pltpu.async_copy(
                out_v.at[b, bb],
                out_hbm.at[pl.ds(bb * S + pos0 + c * P, P)], sem_out[b])

    def wait_out(c, b):
        for bb in range(B):
            pltpu.make_async_copy(
                out_v.at[b, bb],
                out_hbm.at[pl.ds(bb * S + pos0 + c * P, P)], sem_out[b]).wait()

    def add_chunk(b):
        for j in range(P):
            def vbody(i, _, j=j):
                s0 = i * LANES
                vpos = pos_v[b, j, pl.ds(s0, LANES)]
                for bb in range(B):
                    out_v[b, bb, j, pl.ds(s0, LANES)] = (
                        tok_v[b, bb * P + j, pl.ds(s0, LANES)] + vpos)
                return 0
            lax.fori_loop(0, D // LANES, vbody, 0)

    for b in range(NBUF):
        issue_in(b, b)

    def outer(g, _):
        for b in range(NBUF):
            c = g * NBUF + b
            wait_in(c, b)

            @pl.when(c >= NBUF)
            def _():
                wait_out(c - NBUF, b)

            add_chunk(b)
            issue_out(c, b)

            @pl.when(c + NBUF < NCHUNKS)
            def _():
                issue_in(c + NBUF, b)
        return 0

    lax.fori_loop(0, NCHUNKS // NBUF, outer, 0)

    for b in range(NBUF):
        wait_out(NCHUNKS - NBUF + b, b)


def _run(xr, token_emb, pos_emb):
    mesh = plsc.VectorSubcoreMesh(core_axis_name="c", subcore_axis_name="s")
    kern = functools.partial(
        pl.kernel,
        mesh=mesh,
        out_type=jax.ShapeDtypeStruct((FLAT, D), jnp.float32),
        scratch_types=[
            pltpu.VMEM((ROWS_PER_W,), jnp.int32),
            pltpu.VMEM((NBUF, CR, D), jnp.float32),
            pltpu.VMEM((NBUF, P, D), jnp.float32),
            pltpu.VMEM((NBUF, B, P, D), jnp.float32),
            [pltpu.SemaphoreType.DMA] * NBUF,
            [pltpu.SemaphoreType.DMA] * NBUF,
            [pltpu.SemaphoreType.DMA] * NBUF,
        ],
    )(_body)
    return kern(xr, token_emb, pos_emb)


def kernel(x, token_emb, pos_emb):
    xr = (x.astype(jnp.int32).T
          .reshape(NW, NCHUNKS, P, B)
          .transpose(0, 1, 3, 2)
          .reshape(FLAT))
    out = _run(xr, token_emb, pos_emb)
    return out.reshape(B, S, D)


# speedup vs baseline: 1.3350x; 1.3350x over previous
import functools

import jax
import jax.numpy as jnp
from jax import lax
from jax.experimental import pallas as pl
from jax.experimental.pallas import tpu as pltpu
from jax.experimental.pallas import tpu_sc as plsc

B = 4
S = 2048
D = 2048
FLAT = B * S
NC = 2
NS = 16
NW = NC * NS
PW = S // NW
HALF = PW // 2
CH = 8
ROWS_PER_W = PW * B
NCH = ROWS_PER_W // CH
NT = 3
LANES = 16


def _decomp(c):
    return c // (NCH // 2), (c % (NCH // 2)) // 4, c % 4


def _body(xr_hbm, tok_hbm, pos_hbm, out_hbm,
          idx_v, tok_v, pos_res, sem_tok, sem_pos, sem_out):
    wid = lax.axis_index("s") * NC + lax.axis_index("c")
    base = wid * ROWS_PER_W
    pos0 = wid * PW

    pltpu.sync_copy(xr_hbm.at[pl.ds(base, ROWS_PER_W)], idx_v)

    def pos_copy(h):
        return pltpu.make_async_copy(
            pos_hbm.at[pl.ds(pos0 + h * HALF, HALF)], pos_res, sem_pos)

    def gather_copy(c, t):
        return pltpu.make_async_copy(
            tok_hbm.at[idx_v.at[pl.ds(c * CH, CH)]], tok_v.at[t], sem_tok[t])

    def out_copy(c, t):
        h, b, cc = _decomp(c)
        off = b * S + pos0 + h * HALF + cc * CH
        return pltpu.make_async_copy(
            tok_v.at[t], out_hbm.at[pl.ds(off, CH)], sem_out[t])

    def add_chunk(c, t):
        _, _, cc = _decomp(c)

        def vbody(i, _):
            s0 = i * LANES
            for k in range(CH):
                plsc.addupdate(tok_v.at[t, k, pl.ds(s0, LANES)],
                               pos_res[cc * CH + k, pl.ds(s0, LANES)])
            return 0
        lax.fori_loop(0, D // LANES, vbody, 0)

    pos_copy(0).start()
    gather_copy(0, 0).start()
    gather_copy(1, 1).start()
    pos_copy(0).wait()

    for c in range(NCH):
        t = c % NT
        gather_copy(c, t).wait()
        if c == NCH // 2:
            pos_copy(1).wait()
        add_chunk(c, t)
        out_copy(c, t).start()
        if c >= 1:
            out_copy(c - 1, (c - 1) % NT).wait()
        if c + 2 < NCH:
            gather_copy(c + 2, (c + 2) % NT).start()
        if c == NCH // 2 - 1:
            pos_copy(1).start()
    out_copy(NCH - 1, (NCH - 1) % NT).wait()


def _run(xr, token_emb, pos_emb):
    mesh = plsc.VectorSubcoreMesh(core_axis_name="c", subcore_axis_name="s")
    kern = functools.partial(
        pl.kernel,
        mesh=mesh,
        out_type=jax.ShapeDtypeStruct((FLAT, D), jnp.float32),
        scratch_types=[
            pltpu.VMEM((ROWS_PER_W,), jnp.int32),
            pltpu.VMEM((NT, CH, D), jnp.float32),
            pltpu.VMEM((HALF, D), jnp.float32),
            [pltpu.SemaphoreType.DMA] * NT,
            pltpu.SemaphoreType.DMA,
            [pltpu.SemaphoreType.DMA] * NT,
        ],
    )(_body)
    return kern(xr, token_emb, pos_emb)


def kernel(x, token_emb, pos_emb):
    xr = (x.astype(jnp.int32).T
          .reshape(NW, 2, 4, CH, B)
          .transpose(0, 1, 4, 2, 3)
          .reshape(FLAT))
    out = _run(xr, token_emb, pos_emb)
    return out.reshape(B, S, D)


# speedup vs baseline: 1.7247x; 1.2919x over previous
import functools

import jax
import jax.numpy as jnp
from jax import lax
from jax.experimental import pallas as pl
from jax.experimental.pallas import tpu as pltpu
from jax.experimental.pallas import tpu_sc as plsc

B = 4
S = 2048
D = 2048
FLAT = B * S
NC = 2
NS = 16
NW = NC * NS
PW = S // NW
QR = 16
NQ = PW // QR
CH = 8
ROWS_PER_W = PW * B
NCH = ROWS_PER_W // CH
NBUF = 2
LANES = 16


def _body(xr_hbm, tok_hbm, pos_hbm, out_hbm,
          idx_v, tok_v, pos_res, out_v, sem_tok, sem_pos, sem_out):
    wid = lax.axis_index("s") * NC + lax.axis_index("c")
    base = wid * ROWS_PER_W
    pos0 = wid * PW

    pltpu.sync_copy(xr_hbm.at[pl.ds(base, ROWS_PER_W)], idx_v)

    def pos_copy(q):
        return pltpu.make_async_copy(
            pos_hbm.at[pl.ds(pos0 + q * QR, QR)], pos_res, sem_pos)

    def gather_copy(c, t):
        return pltpu.make_async_copy(
            tok_hbm.at[idx_v.at[pl.ds(c * CH, CH)]], tok_v.at[t], sem_tok[t])

    def out_copy(c, t):
        q = c // 8
        b = (c % 8) // 2
        off = b * S + pos0 + q * QR + t * CH
        return pltpu.make_async_copy(
            out_v.at[t], out_hbm.at[pl.ds(off, CH)], sem_out[t])

    def add_chunk(t):
        def vbody(i, _):
            s0 = i * LANES
            for k in range(CH):
                out_v[t, k, pl.ds(s0, LANES)] = (
                    tok_v[t, k, pl.ds(s0, LANES)]
                    + pos_res[t * CH + k, pl.ds(s0, LANES)])
            return 0
        lax.fori_loop(0, D // LANES, vbody, 0)

    pos_copy(0).start()
    gather_copy(0, 0).start()
    gather_copy(1, 1).start()
    pos_copy(0).wait()

    def step(g, _):
        for t in range(NBUF):
            c = g * NBUF + t

            @pl.when(jnp.logical_and(c % 8 == 0, c > 0))
            def _():
                pos_copy(c // 8).wait()

            gather_copy(c, t).wait()

            @pl.when(c >= NBUF)
            def _():
                out_copy(c - NBUF, t).wait()

            add_chunk(t)

            @pl.when(c + NBUF < NCH)
            def _():
                gather_copy(c + NBUF, t).start()

            out_copy(c, t).start()

            @pl.when(jnp.logical_and(c % 8 == 7, c < NCH - 1))
            def _():
                pos_copy(c // 8 + 1).start()
        return 0

    lax.fori_loop(0, NCH // NBUF, step, 0)

    for t in range(NBUF):
        out_copy(NCH - NBUF + t, t).wait()


def _run(xr, token_emb, pos_emb):
    mesh = plsc.VectorSubcoreMesh(core_axis_name="c", subcore_axis_name="s")
    kern = functools.partial(
        pl.kernel,
        mesh=mesh,
        out_type=jax.ShapeDtypeStruct((FLAT, D), jnp.float32),
        scratch_types=[
            pltpu.VMEM((ROWS_PER_W,), jnp.int32),
            pltpu.VMEM((NBUF, CH, D), jnp.float32),
            pltpu.VMEM((QR, D), jnp.float32),
            pltpu.VMEM((NBUF, CH, D), jnp.float32),
            [pltpu.SemaphoreType.DMA] * NBUF,
            pltpu.SemaphoreType.DMA,
            [pltpu.SemaphoreType.DMA] * NBUF,
        ],
    )(_body)
    return kern(xr, token_emb, pos_emb)


def kernel(x, token_emb, pos_emb):
    xr = (x.astype(jnp.int32).T
          .reshape(NW, NQ, NBUF, CH, B)
          .transpose(0, 1, 4, 2, 3)
          .reshape(FLAT))
    out = _run(xr, token_emb, pos_emb)
    return out.reshape(B, S, D)
